# baseline (speedup 1.0000x reference)
import functools

import jax
import jax.numpy as jnp
from jax import lax
from jax.experimental import pallas as pl
from jax.experimental.pallas import tpu as pltpu
from jax.experimental.pallas import tpu_sc as plsc

_TV_WEIGHT = 0.1
_NEG_CLAMP = -100.0


def _sc_point_gather(y_rows, pts_flat):
    total = pts_flat.shape[0] // 2
    n_workers = 32
    npw = total // n_workers
    groups = npw // 16
    W = y_rows.shape[1]

    mesh = plsc.VectorSubcoreMesh(core_axis_name="c", subcore_axis_name="s")

    @functools.partial(
        pl.kernel,
        mesh=mesh,
        out_type=jax.ShapeDtypeStruct((total,), jnp.float32),
        scratch_types=[
            pltpu.VMEM((npw,), jnp.int32),
            pltpu.VMEM((npw,), jnp.int32),
            pltpu.VMEM((npw,), jnp.int32),
            pltpu.VMEM((npw,), jnp.int32),
            pltpu.VMEM((npw, W), jnp.float32),
            pltpu.VMEM((npw,), jnp.float32),
            pltpu.SemaphoreType.DMA,
        ],
        compiler_params=pltpu.CompilerParams(needs_layout_passes=False),
    )
    def k(y_hbm, pts_hbm, out_hbm, ia_v, ib_v, x_v, row_v, rows_v, g_v, sem):
        wid = lax.axis_index("s") * 2 + lax.axis_index("c")
        base = wid * npw
        b = base // 512
        for g in range(groups):
            p = lax.iota(jnp.int32, 16) + (base + g * 16)
            ia_v[pl.ds(g * 16, 16)] = p * 2
            ib_v[pl.ds(g * 16, 16)] = p * 2 + 1
        cpa = pltpu.async_copy(pts_hbm.at[ia_v], x_v, sem)
        cpb = pltpu.async_copy(pts_hbm.at[ib_v], row_v, sem)
        cpa.wait()
        cpb.wait()
        for g in range(groups):
            xs = x_v[pl.ds(g * 16, 16)]
            ys = row_v[pl.ds(g * 16, 16)]
            x_v[pl.ds(g * 16, 16)] = jnp.minimum(jnp.maximum(xs, 0), W - 1)
            ys = jnp.minimum(jnp.maximum(ys, 0), 511)
            row_v[pl.ds(g * 16, 16)] = b * 512 + ys
        pltpu.async_copy(y_hbm.at[row_v], rows_v, sem).wait()
        for g in range(groups):
            rid = lax.iota(jnp.int32, 16) + g * 16
            xs = x_v[pl.ds(g * 16, 16)]
            g_v[pl.ds(g * 16, 16)] = plsc.load_gather(rows_v, [rid, xs])
        pltpu.sync_copy(g_v, out_hbm.at[pl.ds(base, npw)])

    return k(y_rows, pts_flat)


def _tv_body(*refs):
    out_ref = refs[-1]
    total = jnp.float32(0.0)
    for r in refs[:-1]:
        x = r[0]
        p = 1.0 / (1.0 + jnp.exp(-x))
        dh = jnp.sum(jnp.abs(p[:, 1:] - p[:, :-1]))
        dv = jnp.sum(jnp.abs(p[1:, :] - p[:-1, :]))
        total = total + (dh + dv)
    out_ref[0, 0] = total


def _tc_tv_sum(y3d):
    B, H, W = y3d.shape
    specs = [
        pl.BlockSpec((1, H, W), functools.partial(lambda k, i: (k, 0, 0), k))
        for k in range(B)
    ]
    return pl.pallas_call(
        _tv_body,
        grid=(1,),
        in_specs=specs,
        out_specs=pl.BlockSpec(memory_space=pltpu.SMEM),
        out_shape=jax.ShapeDtypeStruct((1, 1), jnp.float32),
    )(*([y3d] * B))


def _combine_body(tv_ref, g_ref, lab_ref, out_ref):
    g = g_ref[...].reshape(lab_ref.shape)
    lab = lab_ref[...]
    p = 1.0 / (1.0 + jnp.exp(-g))
    log_p = jnp.maximum(jnp.log(p), _NEG_CLAMP)
    log_1mp = jnp.maximum(jnp.log(1.0 - p), _NEG_CLAMP)
    bce = -(lab * log_p + (1.0 - lab) * log_1mp)
    bce_mean = jnp.sum(bce) / (lab.shape[0] * lab.shape[1])
    out_ref[0, 0] = bce_mean + _TV_WEIGHT * tv_ref[0, 0] / lab.shape[0]


def _tc_combine(tv, gathered, labels):
    return pl.pallas_call(
        _combine_body,
        in_specs=[
            pl.BlockSpec(memory_space=pltpu.VMEM),
            pl.BlockSpec(memory_space=pltpu.VMEM),
            pl.BlockSpec(memory_space=pltpu.VMEM),
        ],
        out_specs=pl.BlockSpec(memory_space=pltpu.SMEM),
        out_shape=jax.ShapeDtypeStruct((1, 1), jnp.float32),
    )(tv, gathered, labels)


def kernel(y_pred, labels, points_xy):
    B, _, H, W = y_pred.shape
    N = labels.shape[1]
    y3d = y_pred.reshape(B, H, W)
    y_rows = y_pred.reshape(B * H, W)
    pts_flat = points_xy.astype(jnp.int32).reshape(B * N * 2)

    gathered = _sc_point_gather(y_rows, pts_flat)
    tv = _tc_tv_sum(y3d)
    out = _tc_combine(tv, gathered, labels)
    return out[0, 0]


# speedup vs baseline: 1.0838x; 1.0838x over previous
import functools

import jax
import jax.numpy as jnp
from jax import lax
from jax.experimental import pallas as pl
from jax.experimental.pallas import tpu as pltpu
from jax.experimental.pallas import tpu_sc as plsc

_TV_WEIGHT = 0.1
_NEG_CLAMP = -100.0


def _sc_point_gather(y_rows, pts_t):
    _, B, N = pts_t.shape
    total = B * N
    n_workers = 32
    npw = total // n_workers
    groups = npw // 16
    W = y_rows.shape[1]

    mesh = plsc.VectorSubcoreMesh(core_axis_name="c", subcore_axis_name="s")

    @functools.partial(
        pl.kernel,
        mesh=mesh,
        out_type=jax.ShapeDtypeStruct((total,), jnp.float32),
        scratch_types=[
            pltpu.VMEM((B, N), jnp.int32),
            pltpu.VMEM((B, N), jnp.int32),
            pltpu.VMEM((npw,), jnp.int32),
            pltpu.VMEM((npw,), jnp.int32),
            pltpu.VMEM((npw, W), jnp.float32),
            pltpu.VMEM((npw,), jnp.float32),
            pltpu.SemaphoreType.DMA,
        ],
        compiler_params=pltpu.CompilerParams(needs_layout_passes=False),
    )
    def k(y_hbm, pts_hbm, out_hbm, xp_v, yp_v, x_v, row_v, rows_v, g_v, sem):
        wid = lax.axis_index("s") * 2 + lax.axis_index("c")
        base = wid * npw
        b = base // N
        n0 = base - b * N
        cpa = pltpu.async_copy(pts_hbm.at[0], xp_v, sem)
        cpb = pltpu.async_copy(pts_hbm.at[1], yp_v, sem)
        cpa.wait()
        cpb.wait()
        bvec = jnp.full((16,), 0, jnp.int32) + b
        for g in range(groups):
            nvec = lax.iota(jnp.int32, 16) + (n0 + g * 16)
            xs = plsc.load_gather(xp_v, [bvec, nvec])
            ys = plsc.load_gather(yp_v, [bvec, nvec])
            x_v[pl.ds(g * 16, 16)] = jnp.minimum(jnp.maximum(xs, 0), W - 1)
            ys = jnp.minimum(jnp.maximum(ys, 0), 511)
            row_v[pl.ds(g * 16, 16)] = b * 512 + ys
        pltpu.async_copy(y_hbm.at[row_v], rows_v, sem).wait()
        for g in range(groups):
            rid = lax.iota(jnp.int32, 16) + g * 16
            xs = x_v[pl.ds(g * 16, 16)]
            g_v[pl.ds(g * 16, 16)] = plsc.load_gather(rows_v, [rid, xs])
        pltpu.sync_copy(g_v, out_hbm.at[pl.ds(base, npw)])

    return k(y_rows, pts_t)


def _tv_body(y0_ref, y1_ref, y2_ref, y3_ref, out_ref):
    i = pl.program_id(0)
    total = jnp.float32(0.0)
    for r in (y0_ref, y1_ref, y2_ref, y3_ref):
        x = r[0]
        p = 1.0 / (1.0 + jnp.exp(-x))
        dh = jnp.sum(jnp.abs(p[:, 1:] - p[:, :-1]))
        dv = jnp.sum(jnp.abs(p[1:, :] - p[:-1, :]))
        total = total + (dh + dv)

    @pl.when(i == 0)
    def _():
        out_ref[...] = jnp.zeros_like(out_ref)

    out_ref[...] += total


def _tc_tv_sum(y3d):
    B, H, W = y3d.shape
    specs = [
        pl.BlockSpec((1, H, W), functools.partial(lambda k, i: (i * 4 + k, 0, 0), k))
        for k in range(4)
    ]
    return pl.pallas_call(
        _tv_body,
        grid=(B // 4,),
        in_specs=specs,
        out_specs=pl.BlockSpec((1, 1), lambda i: (0, 0)),
        out_shape=jax.ShapeDtypeStruct((1, 1), jnp.float32),
    )(y3d, y3d, y3d, y3d)


def _combine_body(tv_ref, g_ref, lab_ref, out_ref):
    g = g_ref[...].reshape(lab_ref.shape)
    lab = lab_ref[...]
    p = 1.0 / (1.0 + jnp.exp(-g))
    log_p = jnp.maximum(jnp.log(p), _NEG_CLAMP)
    log_1mp = jnp.maximum(jnp.log(1.0 - p), _NEG_CLAMP)
    bce = -(lab * log_p + (1.0 - lab) * log_1mp)
    bce_mean = jnp.sum(bce) / (lab.shape[0] * lab.shape[1])
    out_ref[0, 0] = bce_mean + _TV_WEIGHT * tv_ref[0, 0] / lab.shape[0]


def _tc_combine(tv, gathered, labels):
    return pl.pallas_call(
        _combine_body,
        in_specs=[
            pl.BlockSpec(memory_space=pltpu.VMEM),
            pl.BlockSpec(memory_space=pltpu.VMEM),
            pl.BlockSpec(memory_space=pltpu.VMEM),
        ],
        out_specs=pl.BlockSpec(memory_space=pltpu.SMEM),
        out_shape=jax.ShapeDtypeStruct((1, 1), jnp.float32),
    )(tv, gathered, labels)


def kernel(y_pred, labels, points_xy):
    B, _, H, W = y_pred.shape
    N = labels.shape[1]
    y3d = y_pred.reshape(B, H, W)
    y_rows = y_pred.reshape(B * H, W)
    pts_t = jnp.transpose(points_xy.astype(jnp.int32), (2, 0, 1))

    gathered = _sc_point_gather(y_rows, pts_t)
    tv = _tc_tv_sum(y3d)
    out = _tc_combine(tv, gathered, labels)
    return out[0, 0]
